# baseline (speedup 1.0000x reference)
import functools

import jax
import jax.numpy as jnp
from jax.experimental import pallas as pl
from jax.experimental.pallas import tpu as pltpu


def _fc_kernel(slot_ref, vb_ref, inp_ref, out_ref, *, scale):
    slot = slot_ref[0]
    total = jnp.sum(vb_ref[...], axis=0)
    slot_row = vb_ref[pl.ds(slot, 1), :, :][0]
    out_ref[...] = (total - slot_row + inp_ref[...]) * scale


def kernel(input, vb, tb, eb, v_next, ptr):
    del tb, eb, v_next
    S, B, D = vb.shape
    slot = jnp.asarray(ptr, jnp.int32) % S
    bb = 256
    grid = (B // bb,)
    body = functools.partial(_fc_kernel, scale=1.0 / S)
    fc = pl.pallas_call(
        body,
        grid_spec=pltpu.PrefetchScalarGridSpec(
            num_scalar_prefetch=1,
            grid=grid,
            in_specs=[
                pl.BlockSpec((S, bb, D), lambda i, slot_ref: (0, i, 0)),
                pl.BlockSpec((bb, D), lambda i, slot_ref: (i, 0)),
            ],
            out_specs=pl.BlockSpec((bb, D), lambda i, slot_ref: (i, 0)),
        ),
        out_shape=jax.ShapeDtypeStruct((B, D), jnp.float32),
    )(slot.reshape((1,)), vb, input)
    return fc
